# baseline (device time: 102741 ns/iter reference)
import jax
import jax.numpy as jnp
from jax import lax
from jax.experimental import pallas as pl
from jax.experimental.pallas import tpu as pltpu

N_DEV = 32
N_EXP = 128
CAP = 6
EXP_PER = N_EXP // N_DEV
SLOTS = EXP_PER * CAP
N_TOK = 1024
TOK_PER = N_TOK // N_DEV


def _ring_allgather(c_block):
    slots, h = c_block.shape

    def body(c_ref, out_ref, send_sems, recv_sems):
        my = lax.axis_index("i")
        left = (my - 1) % N_DEV
        right = (my + 1) % N_DEV

        barrier = pltpu.get_barrier_semaphore()
        for nbr in (left, right):
            pl.semaphore_signal(
                barrier, inc=1,
                device_id=(nbr,), device_id_type=pl.DeviceIdType.MESH,
            )
        pl.semaphore_wait(barrier, 2)

        out_ref[pl.ds(my * slots, slots), :] = c_ref[...]

        for hh in range(N_DEV - 1):
            off = ((my - hh) % N_DEV) * slots
            rdma = pltpu.make_async_remote_copy(
                src_ref=out_ref.at[pl.ds(off, slots), :],
                dst_ref=out_ref.at[pl.ds(off, slots), :],
                send_sem=send_sems.at[hh],
                recv_sem=recv_sems.at[hh],
                device_id=(right,),
                device_id_type=pl.DeviceIdType.MESH,
            )
            rdma.start()
            rdma.wait()

    return pl.pallas_call(
        body,
        out_shape=jax.ShapeDtypeStruct((N_DEV * slots, h), c_block.dtype),
        in_specs=[pl.BlockSpec(memory_space=pltpu.VMEM)],
        out_specs=pl.BlockSpec(memory_space=pltpu.VMEM),
        scratch_shapes=[
            pltpu.SemaphoreType.DMA((N_DEV - 1,)),
            pltpu.SemaphoreType.DMA((N_DEV - 1,)),
        ],
        compiler_params=pltpu.CompilerParams(collective_id=0),
    )(c_block)


def kernel(x, router_W, route_idx, expert_W):
    del router_W
    me = lax.axis_index("i")

    e = route_idx[:, 0].astype(jnp.int32)
    onehot = (e[:, None] == jnp.arange(N_EXP, dtype=jnp.int32)[None, :])
    rank = jnp.take_along_axis(
        jnp.cumsum(onehot.astype(jnp.int32), axis=0), e[:, None], axis=1
    )[:, 0] - 1
    kept = rank < CAP

    e_loc = e - me * EXP_PER
    mine = kept & (e_loc >= 0) & (e_loc < EXP_PER)
    slot = e_loc * CAP + rank
    slot_safe = jnp.where(mine, slot, SLOTS)
    toks = jnp.arange(N_TOK, dtype=jnp.int32)
    tok_of_slot = jnp.zeros(SLOTS + 1, jnp.int32).at[slot_safe].set(toks)[:SLOTS]
    valid = jnp.zeros(SLOTS + 1, jnp.float32).at[slot_safe].set(1.0)[:SLOTS]

    xs = x[tok_of_slot] * valid[:, None]
    xs = xs.astype(jnp.bfloat16).reshape(EXP_PER, CAP, -1)
    w = expert_W.astype(jnp.bfloat16)
    c = jnp.einsum("ecd,edh->ech", xs, w, preferred_element_type=jnp.float32)
    c = c.astype(jnp.bfloat16).reshape(SLOTS, -1)

    g = _ring_allgather(c)

    t_mine = me * TOK_PER + jnp.arange(TOK_PER, dtype=jnp.int32)
    e_m = e[t_mine]
    r_m = jnp.clip(rank[t_mine], 0, CAP - 1)
    flat = (e_m // EXP_PER) * SLOTS + (e_m % EXP_PER) * CAP + r_m
    out = g[flat].astype(jnp.float32)
    return out * kept[t_mine][:, None].astype(jnp.float32)


# device time: 67664 ns/iter; 1.5184x vs baseline; 1.5184x over previous
import jax
import jax.numpy as jnp
from jax import lax
from jax.experimental import pallas as pl
from jax.experimental.pallas import tpu as pltpu

N_DEV = 32
N_EXP = 128
CAP = 6
EXP_PER = N_EXP // N_DEV
SLOTS = EXP_PER * CAP
N_TOK = 1024
TOK_PER = N_TOK // N_DEV


N_STAGES = N_DEV.bit_length() - 1


def _rd_allgather(c_block):
    slots, h = c_block.shape

    def body(c_ref, out_ref, send_sems, recv_sems):
        my = lax.axis_index("i")

        barrier = pltpu.get_barrier_semaphore()
        for k in range(N_STAGES):
            pl.semaphore_signal(
                barrier, inc=1,
                device_id=(my ^ (1 << k),),
                device_id_type=pl.DeviceIdType.MESH,
            )
        pl.semaphore_wait(barrier, N_STAGES)

        out_ref[pl.ds(my * slots, slots), :] = c_ref[...]

        rdmas = []
        for k in range(N_STAGES):
            size = 1 << k
            base = (my // size) * size
            rdma = pltpu.make_async_remote_copy(
                src_ref=out_ref.at[pl.ds(base * slots, size * slots), :],
                dst_ref=out_ref.at[pl.ds(base * slots, size * slots), :],
                send_sem=send_sems.at[k],
                recv_sem=recv_sems.at[k],
                device_id=(my ^ size,),
                device_id_type=pl.DeviceIdType.MESH,
            )
            rdma.start()
            rdma.wait_recv()
            rdmas.append(rdma)
        for rdma in rdmas:
            rdma.wait_send()

    return pl.pallas_call(
        body,
        out_shape=jax.ShapeDtypeStruct((N_DEV * slots, h), c_block.dtype),
        in_specs=[pl.BlockSpec(memory_space=pltpu.VMEM)],
        out_specs=pl.BlockSpec(memory_space=pltpu.VMEM),
        scratch_shapes=[
            pltpu.SemaphoreType.DMA((N_STAGES,)),
            pltpu.SemaphoreType.DMA((N_STAGES,)),
        ],
        compiler_params=pltpu.CompilerParams(collective_id=0),
    )(c_block)


def kernel(x, router_W, route_idx, expert_W):
    del router_W
    me = lax.axis_index("i")

    e = route_idx[:, 0].astype(jnp.int32)
    onehot = (e[:, None] == jnp.arange(N_EXP, dtype=jnp.int32)[None, :])
    rank = jnp.take_along_axis(
        jnp.cumsum(onehot.astype(jnp.int32), axis=0), e[:, None], axis=1
    )[:, 0] - 1
    kept = rank < CAP

    e_loc = e - me * EXP_PER
    mine = kept & (e_loc >= 0) & (e_loc < EXP_PER)
    slot = e_loc * CAP + rank
    slot_safe = jnp.where(mine, slot, SLOTS)
    toks = jnp.arange(N_TOK, dtype=jnp.int32)
    tok_of_slot = jnp.zeros(SLOTS + 1, jnp.int32).at[slot_safe].set(toks)[:SLOTS]
    valid = jnp.zeros(SLOTS + 1, jnp.float32).at[slot_safe].set(1.0)[:SLOTS]

    xs = x[tok_of_slot] * valid[:, None]
    xs = xs.astype(jnp.bfloat16).reshape(EXP_PER, CAP, -1)
    w = expert_W.astype(jnp.bfloat16)
    c = jnp.einsum("ecd,edh->ech", xs, w, preferred_element_type=jnp.float32)
    c = c.astype(jnp.bfloat16).reshape(SLOTS, -1)

    g = _rd_allgather(c)

    t_mine = me * TOK_PER + jnp.arange(TOK_PER, dtype=jnp.int32)
    e_m = e[t_mine]
    r_m = jnp.clip(rank[t_mine], 0, CAP - 1)
    flat = (e_m // EXP_PER) * SLOTS + (e_m % EXP_PER) * CAP + r_m
    out = g[flat].astype(jnp.float32)
    return out * kept[t_mine][:, None].astype(jnp.float32)


# device time: 42629 ns/iter; 2.4101x vs baseline; 1.5873x over previous
import jax
import jax.numpy as jnp
from jax import lax
from jax.experimental import pallas as pl
from jax.experimental.pallas import tpu as pltpu

N_DEV = 32
N_EXP = 128
CAP = 6
EXP_PER = N_EXP // N_DEV
SLOTS = EXP_PER * CAP
N_TOK = 1024
TOK_PER = N_TOK // N_DEV


N_STAGES = N_DEV.bit_length() - 1


def _rd_allgather(c_block):
    slots, h = c_block.shape

    def body(c_ref, out_ref, send_sems, recv_sems):
        my = lax.axis_index("i")

        barrier = pltpu.get_barrier_semaphore()
        for k in range(N_STAGES):
            pl.semaphore_signal(
                barrier, inc=1,
                device_id=(my ^ (1 << k),),
                device_id_type=pl.DeviceIdType.MESH,
            )
        pl.semaphore_wait(barrier, N_STAGES)

        out_ref[pl.ds(my * slots, slots), :] = c_ref[...]

        rdmas = []
        for k in range(N_STAGES):
            size = 1 << k
            base = (my // size) * size
            rdma = pltpu.make_async_remote_copy(
                src_ref=out_ref.at[pl.ds(base * slots, size * slots), :],
                dst_ref=out_ref.at[pl.ds(base * slots, size * slots), :],
                send_sem=send_sems.at[k],
                recv_sem=recv_sems.at[k],
                device_id=(my ^ size,),
                device_id_type=pl.DeviceIdType.MESH,
            )
            rdma.start()
            rdma.wait_recv()
            rdmas.append(rdma)
        for rdma in rdmas:
            rdma.wait_send()

    return pl.pallas_call(
        body,
        out_shape=jax.ShapeDtypeStruct((N_DEV * slots, h), c_block.dtype),
        in_specs=[pl.BlockSpec(memory_space=pltpu.VMEM)],
        out_specs=pl.BlockSpec(memory_space=pltpu.VMEM),
        scratch_shapes=[
            pltpu.SemaphoreType.DMA((N_STAGES,)),
            pltpu.SemaphoreType.DMA((N_STAGES,)),
        ],
        compiler_params=pltpu.CompilerParams(collective_id=0),
    )(c_block)


PAD = 8
SLOTS_P = EXP_PER * PAD
D = 512


def _direct_moe(xs_bf, w_bf, valid, dest_chip, dest_row, kept_m):
    h = w_bf.shape[-1]

    def body(valid_ref, dchip_ref, drow_ref, kept_ref,
             xs_ref, w_ref, out_ref, c_ref, send_sems, recv_sems):
        my = lax.axis_index("i")

        out_ref[...] = jnp.zeros_like(out_ref)

        barrier = pltpu.get_barrier_semaphore()
        for off in range(1, N_DEV):
            pl.semaphore_signal(
                barrier, inc=1,
                device_id=((my + off) % N_DEV,),
                device_id_type=pl.DeviceIdType.MESH,
            )
        pl.semaphore_wait(barrier, N_DEV - 1)

        for le in range(EXP_PER):
            c_ref[pl.ds(le * PAD, PAD), :] = jnp.dot(
                xs_ref[pl.ds(le * PAD, PAD), :],
                w_ref[le],
                preferred_element_type=jnp.float32,
            )

        for s in range(SLOTS_P):
            @pl.when(valid_ref[s] != 0)
            def _send(s=s):
                row = drow_ref[s]
                pltpu.make_async_remote_copy(
                    src_ref=c_ref.at[pl.ds(s, 1), :],
                    dst_ref=out_ref.at[pl.ds(row, 1), :],
                    send_sem=send_sems.at[s],
                    recv_sem=recv_sems.at[row],
                    device_id=(dchip_ref[s],),
                    device_id_type=pl.DeviceIdType.MESH,
                ).start()

        for j in range(TOK_PER):
            @pl.when(kept_ref[j] != 0)
            def _recv(j=j):
                pltpu.make_async_remote_copy(
                    src_ref=c_ref.at[pl.ds(0, 1), :],
                    dst_ref=out_ref.at[pl.ds(j, 1), :],
                    send_sem=send_sems.at[0],
                    recv_sem=recv_sems.at[j],
                    device_id=(my,),
                    device_id_type=pl.DeviceIdType.MESH,
                ).wait_recv()

        for s in range(SLOTS_P):
            @pl.when(valid_ref[s] != 0)
            def _drain(s=s):
                pltpu.make_async_remote_copy(
                    src_ref=c_ref.at[pl.ds(s, 1), :],
                    dst_ref=out_ref.at[pl.ds(0, 1), :],
                    send_sem=send_sems.at[s],
                    recv_sem=recv_sems.at[0],
                    device_id=(my,),
                    device_id_type=pl.DeviceIdType.MESH,
                ).wait_send()

        for off in range(1, N_DEV):
            pl.semaphore_signal(
                barrier, inc=1,
                device_id=((my + off) % N_DEV,),
                device_id_type=pl.DeviceIdType.MESH,
            )
        pl.semaphore_wait(barrier, N_DEV - 1)

    return pl.pallas_call(
        body,
        out_shape=jax.ShapeDtypeStruct((TOK_PER, h), jnp.float32),
        in_specs=[
            pl.BlockSpec(memory_space=pltpu.SMEM),
            pl.BlockSpec(memory_space=pltpu.SMEM),
            pl.BlockSpec(memory_space=pltpu.SMEM),
            pl.BlockSpec(memory_space=pltpu.SMEM),
            pl.BlockSpec(memory_space=pltpu.VMEM),
            pl.BlockSpec(memory_space=pltpu.VMEM),
        ],
        out_specs=pl.BlockSpec(memory_space=pltpu.VMEM),
        scratch_shapes=[
            pltpu.VMEM((SLOTS_P, h), jnp.float32),
            pltpu.SemaphoreType.DMA((SLOTS_P,)),
            pltpu.SemaphoreType.DMA((TOK_PER,)),
        ],
        compiler_params=pltpu.CompilerParams(collective_id=0),
    )(valid, dest_chip, dest_row, kept_m, xs_bf, w_bf)


def kernel(x, router_W, route_idx, expert_W):
    del router_W
    me = lax.axis_index("i")

    e = route_idx[:, 0].astype(jnp.int32)
    onehot = (e[:, None] == jnp.arange(N_EXP, dtype=jnp.int32)[None, :])
    rank = jnp.take_along_axis(
        jnp.cumsum(onehot.astype(jnp.int32), axis=0), e[:, None], axis=1
    )[:, 0] - 1
    kept = rank < CAP

    e_loc = e - me * EXP_PER
    mine = kept & (e_loc >= 0) & (e_loc < EXP_PER)
    slot = e_loc * PAD + rank
    slot_safe = jnp.where(mine, slot, SLOTS_P)
    toks = jnp.arange(N_TOK, dtype=jnp.int32)
    tok_of_slot = jnp.zeros(SLOTS_P + 1, jnp.int32).at[slot_safe].set(toks)[:SLOTS_P]
    valid = jnp.zeros(SLOTS_P + 1, jnp.int32).at[slot_safe].set(1)[:SLOTS_P]
    dest_chip = tok_of_slot // TOK_PER
    dest_row = tok_of_slot % TOK_PER

    t_mine = me * TOK_PER + jnp.arange(TOK_PER, dtype=jnp.int32)
    kept_m = kept[t_mine].astype(jnp.int32)

    xs_bf = x[tok_of_slot].astype(jnp.bfloat16)
    return _direct_moe(
        xs_bf,
        expert_W.astype(jnp.bfloat16),
        valid, dest_chip, dest_row, kept_m,
    )


# device time: 33646 ns/iter; 3.0536x vs baseline; 1.2670x over previous
import jax
import jax.numpy as jnp
from jax import lax
from jax.experimental import pallas as pl
from jax.experimental.pallas import tpu as pltpu

N_DEV = 32
N_EXP = 128
CAP = 6
EXP_PER = N_EXP // N_DEV
SLOTS = EXP_PER * CAP
N_TOK = 1024
TOK_PER = N_TOK // N_DEV


N_STAGES = N_DEV.bit_length() - 1


def _rd_allgather(c_block):
    slots, h = c_block.shape

    def body(c_ref, out_ref, send_sems, recv_sems):
        my = lax.axis_index("i")

        barrier = pltpu.get_barrier_semaphore()
        for k in range(N_STAGES):
            pl.semaphore_signal(
                barrier, inc=1,
                device_id=(my ^ (1 << k),),
                device_id_type=pl.DeviceIdType.MESH,
            )
        pl.semaphore_wait(barrier, N_STAGES)

        out_ref[pl.ds(my * slots, slots), :] = c_ref[...]

        rdmas = []
        for k in range(N_STAGES):
            size = 1 << k
            base = (my // size) * size
            rdma = pltpu.make_async_remote_copy(
                src_ref=out_ref.at[pl.ds(base * slots, size * slots), :],
                dst_ref=out_ref.at[pl.ds(base * slots, size * slots), :],
                send_sem=send_sems.at[k],
                recv_sem=recv_sems.at[k],
                device_id=(my ^ size,),
                device_id_type=pl.DeviceIdType.MESH,
            )
            rdma.start()
            rdma.wait_recv()
            rdmas.append(rdma)
        for rdma in rdmas:
            rdma.wait_send()

    return pl.pallas_call(
        body,
        out_shape=jax.ShapeDtypeStruct((N_DEV * slots, h), c_block.dtype),
        in_specs=[pl.BlockSpec(memory_space=pltpu.VMEM)],
        out_specs=pl.BlockSpec(memory_space=pltpu.VMEM),
        scratch_shapes=[
            pltpu.SemaphoreType.DMA((N_STAGES,)),
            pltpu.SemaphoreType.DMA((N_STAGES,)),
        ],
        compiler_params=pltpu.CompilerParams(collective_id=0),
    )(c_block)


PAD = 8
SLOTS_P = EXP_PER * PAD
D = 512


def _direct_moe(xs, w, valid, dest_chip, dest_row, kept_m):
    h = w.shape[-1]

    def body(valid_ref, dchip_ref, drow_ref, kept_ref,
             xs_ref, w_ref, out_ref, c_ref, send_sems, recv_sems):
        my = lax.axis_index("i")

        out_ref[...] = jnp.zeros_like(out_ref)

        barrier = pltpu.get_barrier_semaphore()
        for off in range(1, N_DEV):
            pl.semaphore_signal(
                barrier, inc=1,
                device_id=((my + off) % N_DEV,),
                device_id_type=pl.DeviceIdType.MESH,
            )
        pl.semaphore_wait(barrier, N_DEV - 1)

        for le in range(EXP_PER):
            c_ref[pl.ds(le * PAD, PAD), :] = jnp.dot(
                xs_ref[pl.ds(le * PAD, PAD), :],
                w_ref[le],
                preferred_element_type=jnp.float32,
            )

        for s in range(SLOTS_P):
            @pl.when(valid_ref[s] != 0)
            def _send(s=s):
                row = drow_ref[s]
                pltpu.make_async_remote_copy(
                    src_ref=c_ref.at[pl.ds(s, 1), :],
                    dst_ref=out_ref.at[pl.ds(row, 1), :],
                    send_sem=send_sems.at[s],
                    recv_sem=recv_sems.at[row],
                    device_id=(dchip_ref[s],),
                    device_id_type=pl.DeviceIdType.MESH,
                ).start()

        for j in range(TOK_PER):
            @pl.when(kept_ref[j] != 0)
            def _recv(j=j):
                pltpu.make_async_remote_copy(
                    src_ref=c_ref.at[pl.ds(0, 1), :],
                    dst_ref=out_ref.at[pl.ds(j, 1), :],
                    send_sem=send_sems.at[0],
                    recv_sem=recv_sems.at[j],
                    device_id=(my,),
                    device_id_type=pl.DeviceIdType.MESH,
                ).wait_recv()

        for s in range(SLOTS_P):
            @pl.when(valid_ref[s] != 0)
            def _drain(s=s):
                pltpu.make_async_remote_copy(
                    src_ref=c_ref.at[pl.ds(s, 1), :],
                    dst_ref=out_ref.at[pl.ds(0, 1), :],
                    send_sem=send_sems.at[s],
                    recv_sem=recv_sems.at[0],
                    device_id=(my,),
                    device_id_type=pl.DeviceIdType.MESH,
                ).wait_send()

        for off in range(1, N_DEV):
            pl.semaphore_signal(
                barrier, inc=1,
                device_id=((my + off) % N_DEV,),
                device_id_type=pl.DeviceIdType.MESH,
            )
        pl.semaphore_wait(barrier, N_DEV - 1)

    return pl.pallas_call(
        body,
        out_shape=jax.ShapeDtypeStruct((TOK_PER, h), jnp.float32),
        in_specs=[
            pl.BlockSpec(memory_space=pltpu.SMEM),
            pl.BlockSpec(memory_space=pltpu.SMEM),
            pl.BlockSpec(memory_space=pltpu.SMEM),
            pl.BlockSpec(memory_space=pltpu.SMEM),
            pl.BlockSpec(memory_space=pltpu.VMEM),
            pl.BlockSpec(memory_space=pltpu.VMEM),
        ],
        out_specs=pl.BlockSpec(memory_space=pltpu.VMEM),
        scratch_shapes=[
            pltpu.VMEM((SLOTS_P, h), jnp.float32),
            pltpu.SemaphoreType.DMA((SLOTS_P,)),
            pltpu.SemaphoreType.DMA((TOK_PER,)),
        ],
        compiler_params=pltpu.CompilerParams(collective_id=0),
    )(valid, dest_chip, dest_row, kept_m, xs, w)


def kernel(x, router_W, route_idx, expert_W):
    del router_W
    me = lax.axis_index("i")

    e = route_idx[:, 0].astype(jnp.int32)
    ti = jnp.arange(N_TOK, dtype=jnp.int32)
    rank = ((e[None, :] == e[:, None]) & (ti[None, :] < ti[:, None])).sum(
        axis=1, dtype=jnp.int32
    )
    kept = rank < CAP

    e_loc = e - me * EXP_PER
    mine = kept & (e_loc >= 0) & (e_loc < EXP_PER)
    slot = e_loc * PAD + rank
    slot_safe = jnp.where(mine, slot, SLOTS_P)
    tok_of_slot = jnp.zeros(SLOTS_P + 1, jnp.int32).at[slot_safe].set(ti)[:SLOTS_P]
    valid = jnp.zeros(SLOTS_P + 1, jnp.int32).at[slot_safe].set(1)[:SLOTS_P]
    dest_chip = tok_of_slot // TOK_PER
    dest_row = tok_of_slot % TOK_PER

    t_mine = me * TOK_PER + jnp.arange(TOK_PER, dtype=jnp.int32)
    kept_m = kept[t_mine].astype(jnp.int32)

    xs = x[tok_of_slot]
    return _direct_moe(xs, expert_W, valid, dest_chip, dest_row, kept_m)


# device time: 26534 ns/iter; 3.8721x vs baseline; 1.2680x over previous
import jax
import jax.numpy as jnp
from jax import lax
from jax.experimental import pallas as pl
from jax.experimental.pallas import tpu as pltpu

N_DEV = 32
N_EXP = 128
CAP = 6
EXP_PER = N_EXP // N_DEV
SLOTS = EXP_PER * CAP
N_TOK = 1024
TOK_PER = N_TOK // N_DEV


N_STAGES = N_DEV.bit_length() - 1


def _rd_allgather(c_block):
    slots, h = c_block.shape

    def body(c_ref, out_ref, send_sems, recv_sems):
        my = lax.axis_index("i")

        barrier = pltpu.get_barrier_semaphore()
        for k in range(N_STAGES):
            pl.semaphore_signal(
                barrier, inc=1,
                device_id=(my ^ (1 << k),),
                device_id_type=pl.DeviceIdType.MESH,
            )
        pl.semaphore_wait(barrier, N_STAGES)

        out_ref[pl.ds(my * slots, slots), :] = c_ref[...]

        rdmas = []
        for k in range(N_STAGES):
            size = 1 << k
            base = (my // size) * size
            rdma = pltpu.make_async_remote_copy(
                src_ref=out_ref.at[pl.ds(base * slots, size * slots), :],
                dst_ref=out_ref.at[pl.ds(base * slots, size * slots), :],
                send_sem=send_sems.at[k],
                recv_sem=recv_sems.at[k],
                device_id=(my ^ size,),
                device_id_type=pl.DeviceIdType.MESH,
            )
            rdma.start()
            rdma.wait_recv()
            rdmas.append(rdma)
        for rdma in rdmas:
            rdma.wait_send()

    return pl.pallas_call(
        body,
        out_shape=jax.ShapeDtypeStruct((N_DEV * slots, h), c_block.dtype),
        in_specs=[pl.BlockSpec(memory_space=pltpu.VMEM)],
        out_specs=pl.BlockSpec(memory_space=pltpu.VMEM),
        scratch_shapes=[
            pltpu.SemaphoreType.DMA((N_STAGES,)),
            pltpu.SemaphoreType.DMA((N_STAGES,)),
        ],
        compiler_params=pltpu.CompilerParams(collective_id=0),
    )(c_block)


PAD = 8
SLOTS_P = EXP_PER * PAD
D = 512


def _direct_moe(xs, w, valid, dest_chip, dest_row, kept_m):
    h = w.shape[-1]

    def body(valid_ref, dchip_ref, drow_ref, kept_ref,
             xs_ref, w_hbm_ref, out_ref, w_ref, c_ref,
             w_sem, send_sems, recv_sems):
        my = lax.axis_index("i")

        w_copy = pltpu.make_async_copy(w_hbm_ref, w_ref, w_sem)
        w_copy.start()

        out_ref[...] = jnp.zeros_like(out_ref)

        barrier = pltpu.get_barrier_semaphore()
        for off in range(1, N_DEV):
            pl.semaphore_signal(
                barrier, inc=1,
                device_id=((my + off) % N_DEV,),
                device_id_type=pl.DeviceIdType.MESH,
            )
        pl.semaphore_wait(barrier, N_DEV - 1)
        w_copy.wait()

        for le in range(EXP_PER):
            c_ref[pl.ds(le * PAD, PAD), :] = jnp.dot(
                xs_ref[pl.ds(le * PAD, PAD), :],
                w_ref[le],
                preferred_element_type=jnp.float32,
            )

        for s in range(SLOTS_P):
            @pl.when(valid_ref[s] != 0)
            def _send(s=s):
                row = drow_ref[s]
                pltpu.make_async_remote_copy(
                    src_ref=c_ref.at[pl.ds(s, 1), :],
                    dst_ref=out_ref.at[pl.ds(row, 1), :],
                    send_sem=send_sems.at[s],
                    recv_sem=recv_sems.at[row],
                    device_id=(dchip_ref[s],),
                    device_id_type=pl.DeviceIdType.MESH,
                ).start()

        for j in range(TOK_PER):
            @pl.when(kept_ref[j] != 0)
            def _recv(j=j):
                pltpu.make_async_remote_copy(
                    src_ref=c_ref.at[pl.ds(0, 1), :],
                    dst_ref=out_ref.at[pl.ds(j, 1), :],
                    send_sem=send_sems.at[0],
                    recv_sem=recv_sems.at[j],
                    device_id=(my,),
                    device_id_type=pl.DeviceIdType.MESH,
                ).wait_recv()

        for s in range(SLOTS_P):
            @pl.when(valid_ref[s] != 0)
            def _drain(s=s):
                pltpu.make_async_remote_copy(
                    src_ref=c_ref.at[pl.ds(s, 1), :],
                    dst_ref=out_ref.at[pl.ds(0, 1), :],
                    send_sem=send_sems.at[s],
                    recv_sem=recv_sems.at[0],
                    device_id=(my,),
                    device_id_type=pl.DeviceIdType.MESH,
                ).wait_send()

        for off in range(1, N_DEV):
            pl.semaphore_signal(
                barrier, inc=1,
                device_id=((my + off) % N_DEV,),
                device_id_type=pl.DeviceIdType.MESH,
            )
        pl.semaphore_wait(barrier, N_DEV - 1)

    return pl.pallas_call(
        body,
        out_shape=jax.ShapeDtypeStruct((TOK_PER, h), jnp.float32),
        in_specs=[
            pl.BlockSpec(memory_space=pltpu.SMEM),
            pl.BlockSpec(memory_space=pltpu.SMEM),
            pl.BlockSpec(memory_space=pltpu.SMEM),
            pl.BlockSpec(memory_space=pltpu.SMEM),
            pl.BlockSpec(memory_space=pltpu.VMEM),
            pl.BlockSpec(memory_space=pltpu.MemorySpace.HBM),
        ],
        out_specs=pl.BlockSpec(memory_space=pltpu.VMEM),
        scratch_shapes=[
            pltpu.VMEM(w.shape, jnp.float32),
            pltpu.VMEM((SLOTS_P, h), jnp.float32),
            pltpu.SemaphoreType.DMA(()),
            pltpu.SemaphoreType.DMA((SLOTS_P,)),
            pltpu.SemaphoreType.DMA((TOK_PER,)),
        ],
        compiler_params=pltpu.CompilerParams(collective_id=0),
    )(valid, dest_chip, dest_row, kept_m, xs, w)


def kernel(x, router_W, route_idx, expert_W):
    del router_W
    me = lax.axis_index("i")

    e = route_idx[:, 0].astype(jnp.int32)
    ti = jnp.arange(N_TOK, dtype=jnp.int32)
    rank = ((e[None, :] == e[:, None]) & (ti[None, :] < ti[:, None])).sum(
        axis=1, dtype=jnp.int32
    )
    kept = rank < CAP

    e_loc = e - me * EXP_PER
    mine = kept & (e_loc >= 0) & (e_loc < EXP_PER)
    slot = e_loc * PAD + rank
    srange = jnp.arange(SLOTS_P, dtype=jnp.int32)
    hit = mine[None, :] & (slot[None, :] == srange[:, None])
    tok_of_slot = (hit * ti[None, :]).sum(axis=1, dtype=jnp.int32)
    valid = hit.sum(axis=1, dtype=jnp.int32)
    dest_chip = tok_of_slot // TOK_PER
    dest_row = tok_of_slot % TOK_PER

    t_mine = me * TOK_PER + jnp.arange(TOK_PER, dtype=jnp.int32)
    kept_m = kept[t_mine].astype(jnp.int32)

    xs = x[tok_of_slot]
    return _direct_moe(xs, expert_W, valid, dest_chip, dest_row, kept_m)
